# baseline (speedup 1.0000x reference)
import functools

import jax
import jax.numpy as jnp
from jax import lax
from jax.experimental import pallas as pl
from jax.experimental.pallas import tpu as pltpu
from jax.experimental.pallas import tpu_sc as plsc

HID_W = 64
HID_P = 32
HID_S = 32
HID = HID_W + HID_P + HID_S

GRP = 128
SUP = 8
NSUB = 16
NCORE = 2
LANES = 16


def _make_kernel(n_tokens: int, n_shape: int):
    nw = NSUB * NCORE
    per_w = n_tokens // nw
    n_grp = per_w // GRP
    nset = 4
    n_quad = n_grp // nset
    n_chunk = n_grp // SUP

    mesh = plsc.VectorSubcoreMesh(core_axis_name="c", subcore_axis_name="s")

    @functools.partial(
        pl.kernel,
        mesh=mesh,
        out_type=jax.ShapeDtypeStruct((n_tokens, HID), jnp.float32),
        scratch_types=[
            pltpu.VMEM((n_grp, GRP), jnp.int32),
            pltpu.VMEM((n_grp, GRP), jnp.int32),
            pltpu.VMEM((SUP, GRP), jnp.int32),
        ] + [pltpu.VMEM((GRP, HID), jnp.float32)] * 4
          + [pltpu.SemaphoreType.DMA] * 4
          + [pltpu.SemaphoreType.DMA] * 4
          + [pltpu.SemaphoreType.DMA] * 4,
        compiler_params=pltpu.CompilerParams(needs_layout_passes=False),
    )
    def k(word_hbm, ps_hbm, catid_hbm, posid_hbm, shpid_hbm,
          out_hbm, idw, idf, tmp, *sets):
        bufs = sets[0:4]
        gw = sets[4:8]
        gp = sets[8:12]
        ss = sets[12:16]
        cid = lax.axis_index("c")
        sid = lax.axis_index("s")
        wid = sid * NCORE + cid
        row0 = wid * n_grp
        base0 = wid * per_w

        pltpu.sync_copy(catid_hbm.at[pl.ds(row0, n_grp)], idw)
        pltpu.sync_copy(posid_hbm.at[pl.ds(row0, n_grp)], idf)

        def fuse(c, _):
            pltpu.sync_copy(shpid_hbm.at[pl.ds(row0 + c * SUP, SUP)], tmp)
            for r in range(SUP):
                for g in range(GRP // LANES):
                    sl = pl.ds(g * LANES, LANES)
                    idf[c * SUP + r, sl] = (
                        idf[c * SUP + r, sl] * n_shape + tmp[r, sl])
            return ()

        lax.fori_loop(0, n_chunk, fuse, (), unroll=False)

        def wgather(j, buf, sem):
            pltpu.async_copy(word_hbm.at[idw.at[j]], buf, sem)

        def pgather(j, buf, sem):
            pltpu.async_copy(ps_hbm.at[idf.at[j]], buf, sem, add=True)

        def gwait(src, buf, sem):
            pltpu.make_async_copy(src.at[pl.ds(0, GRP)], buf, sem).wait()

        def store(j, buf, sem):
            pltpu.async_copy(buf, out_hbm.at[pl.ds(base0 + j * GRP, GRP)],
                             sem)

        def swait(j, buf, sem):
            pltpu.make_async_copy(buf,
                                  out_hbm.at[pl.ds(base0 + j * GRP, GRP)],
                                  sem).wait()

        for i in range(4):
            wgather(i, bufs[i], gw[i])

        def body(m, _):
            j0 = 4 * m
            for i in range(4):
                gwait(word_hbm, bufs[i], gw[i])
                pgather(j0 + i, bufs[i], gp[i])
            for i in range(4):
                gwait(ps_hbm, bufs[i], gp[i])
                store(j0 + i, bufs[i], ss[i])
            for i in range(4):
                swait(j0 + i, bufs[i], ss[i])

                @pl.when(m < n_quad - 1)
                def _(i=i):
                    wgather(j0 + i + 4, bufs[i], gw[i])

            return ()

        lax.fori_loop(0, n_quad, body, (), unroll=False)

    return k


def kernel(word_table, pos_table, shape_table, cat_ids, position_ids, shape_ids):
    b, l = cat_ids.shape
    n = b * l
    n_pos = pos_table.shape[0]
    n_shape = shape_table.shape[0]
    vocab, hw = word_table.shape
    word_pad = jnp.concatenate(
        [word_table, jnp.zeros((vocab, HID - hw), word_table.dtype)], axis=1)
    ps_tab = jnp.concatenate(
        [jnp.zeros((n_pos * n_shape, HID_W), jnp.float32),
         jnp.repeat(pos_table, n_shape, axis=0),
         jnp.tile(shape_table, (n_pos, 1))], axis=1)
    cat2d = cat_ids.reshape(n // GRP, GRP).astype(jnp.int32)
    pos2d = position_ids.reshape(n // GRP, GRP).astype(jnp.int32)
    shp2d = shape_ids.reshape(n // GRP, GRP).astype(jnp.int32)
    k = _make_kernel(n, n_shape)
    out = k(word_pad, ps_tab, cat2d, pos2d, shp2d)
    return out.reshape(b, l, HID)
